# Initial kernel scaffold; baseline (speedup 1.0000x reference)
import jax
import jax.numpy as jnp
from jax.experimental import pallas as pl


def kernel(hidden_states, gate_weight, e_score_correction_bias, w_gate, w_up, w_down, s_gate, s_up, s_down):
    raise NotImplementedError("write your pallas kernel here")


# speedup vs baseline: 1.6419x; 1.6419x over previous
import jax
import jax.numpy as jnp
from jax.experimental import pallas as pl
from jax.experimental.pallas import tpu as pltpu

_N_GROUP = 4
_TOPK_GROUP = 2
_TOP_K = 2
_SCALE = 2.5


def _sigmoid(x):
    return 1.0 / (1.0 + jnp.exp(-x))


def _dot(a, b):
    return jax.lax.dot_general(a, b, (((1,), (0,)), ((), ())),
                               preferred_element_type=jnp.float32)


def _dot_nt(a, b):
    return jax.lax.dot_general(a, b, (((1,), (1,)), ((), ())),
                               preferred_element_type=jnp.float32)


def _routing_cw(logits, bias_row, n_experts):
    f32 = jnp.float32
    scores = _sigmoid(logits)
    swb = scores + bias_row
    cols = [swb[:, j:j + 1] for j in range(n_experts)]
    gsz = n_experts // _N_GROUP
    grp = [sum(cols[g * gsz + u] for u in range(1, gsz)) + cols[g * gsz]
           for g in range(_N_GROUP)]
    selg = []
    for j in range(_N_GROUP):
        r = jnp.zeros_like(grp[j])
        for k in range(_N_GROUP):
            if k == j:
                continue
            beats = (grp[k] >= grp[j]) if k < j else (grp[k] > grp[j])
            r = r + beats.astype(f32)
        selg.append(r < float(_TOPK_GROUP))
    vals = [jnp.where(selg[j // gsz], cols[j], 0.0) for j in range(n_experts)]
    sel = []
    for j in range(n_experts):
        r = jnp.zeros_like(vals[j])
        for k in range(n_experts):
            if k == j:
                continue
            beats = (vals[k] >= vals[j]) if k < j else (vals[k] > vals[j])
            r = r + beats.astype(f32)
        sel.append(r < float(_TOP_K))
    sm = [jnp.where(sel[j], scores[:, j:j + 1], 0.0) for j in range(n_experts)]
    ssum = sm[0] + sm[1]
    for j in range(2, n_experts):
        ssum = ssum + sm[j]
    ssum = ssum + 1e-20
    return jnp.concatenate([s / ssum * _SCALE for s in sm], axis=1)


def _moe_body(xb_ref, gw_ref, bias_ref, wg_ref, wu_ref, wd_ref,
              sg_ref, su_ref, sd_ref, out_ref, cw_ref):
    e = pl.program_id(1)
    n_experts = pl.num_programs(1)
    xb = xb_ref[...]
    bf = jnp.bfloat16

    @pl.when(e == 0)
    def _routing():
        logits = _dot_nt(xb, gw_ref[...])
        cw_ref[...] = _routing_cw(logits, bias_ref[...], n_experts)

    @pl.when(e == 0)
    def _shared():
        s1 = _dot(xb, sg_ref[...])
        s2 = _dot(xb, su_ref[...])
        sact = (s1 * _sigmoid(s1)) * s2
        out_ref[...] = _dot(sact.astype(bf), sd_ref[...])

    h1 = _dot(xb, wg_ref[0])
    h2 = _dot(xb, wu_ref[0])
    act = (h1 * _sigmoid(h1)) * h2
    y = _dot(act.astype(bf), wd_ref[0])
    onehot = (jax.lax.broadcasted_iota(jnp.int32, (1, n_experts), 1) == e
              ).astype(jnp.float32)
    wcol = jnp.sum(cw_ref[...] * onehot, axis=1, keepdims=True)
    out_ref[...] += wcol * y


def kernel(hidden_states, gate_weight, e_score_correction_bias,
           w_gate, w_up, w_down, s_gate, s_up, s_down, *, interpret=False):
    t, h = hidden_states.shape
    n_experts, _, ff = w_gate.shape
    sff = s_gate.shape[1]
    bt = 512
    nt = t // bt
    bf = jnp.bfloat16

    xb = hidden_states.astype(bf)
    gwb = gate_weight.astype(bf)
    bias2d = e_score_correction_bias.reshape(1, n_experts)
    wgb = w_gate.astype(bf)
    wub = w_up.astype(bf)
    wdb = w_down.astype(bf)
    sgb = s_gate.astype(bf)
    sub = s_up.astype(bf)
    sdb = s_down.astype(bf)

    return pl.pallas_call(
        _moe_body,
        grid=(nt, n_experts),
        in_specs=[
            pl.BlockSpec((bt, h), lambda i, e: (i, 0)),
            pl.BlockSpec((n_experts, h), lambda i, e: (0, 0)),
            pl.BlockSpec((1, n_experts), lambda i, e: (0, 0)),
            pl.BlockSpec((1, h, ff), lambda i, e: (e, 0, 0)),
            pl.BlockSpec((1, h, ff), lambda i, e: (e, 0, 0)),
            pl.BlockSpec((1, ff, h), lambda i, e: (e, 0, 0)),
            pl.BlockSpec((h, sff), lambda i, e: (0, 0)),
            pl.BlockSpec((h, sff), lambda i, e: (0, 0)),
            pl.BlockSpec((sff, h), lambda i, e: (0, 0)),
        ],
        out_specs=pl.BlockSpec((bt, h), lambda i, e: (i, 0)),
        out_shape=jax.ShapeDtypeStruct((t, h), jnp.float32),
        scratch_shapes=[pltpu.VMEM((bt, n_experts), jnp.float32)],
        compiler_params=pltpu.CompilerParams(
            dimension_semantics=("arbitrary", "arbitrary")),
        interpret=interpret,
    )(xb, gwb, bias2d, wgb, wub, wdb, sgb, sub, sdb)
